# baseline (speedup 1.0000x reference)
import functools

import jax
import jax.numpy as jnp
from jax.experimental import pallas as pl
from jax.experimental.pallas import tpu as pltpu

_N = 100000
_D = 128
_BLOCK = 2000


def _sum_body(x_ref, o_ref):
    i = pl.program_id(0)

    @pl.when(i == 0)
    def _init():
        o_ref[...] = jnp.zeros_like(o_ref)

    o_ref[...] += jnp.sum(x_ref[...], axis=0, keepdims=True)


def kernel(x):
    grid = _N // _BLOCK
    return pl.pallas_call(
        _sum_body,
        grid=(grid,),
        in_specs=[pl.BlockSpec((_BLOCK, _D), lambda i: (i, 0))],
        out_specs=pl.BlockSpec((1, _D), lambda i: (0, 0)),
        out_shape=jax.ShapeDtypeStruct((1, _D), jnp.float32),
    )(x)


# speedup vs baseline: 1.7572x; 1.7572x over previous
import functools

import jax
import jax.numpy as jnp
from jax.experimental import pallas as pl
from jax.experimental.pallas import tpu as pltpu

_N = 100000
_D = 128
_BLOCK = 10000


def _sum_body(x_ref, o_ref, acc_ref):
    i = pl.program_id(0)

    @pl.when(i == 0)
    def _init():
        acc_ref[...] = jnp.zeros_like(acc_ref)

    acc_ref[...] += jnp.sum(
        x_ref[...].reshape(_BLOCK // 8, 8, _D), axis=0)

    @pl.when(i == pl.num_programs(0) - 1)
    def _finish():
        o_ref[...] = jnp.sum(acc_ref[...], axis=0, keepdims=True)


def kernel(x):
    grid = _N // _BLOCK
    return pl.pallas_call(
        _sum_body,
        grid=(grid,),
        in_specs=[pl.BlockSpec((_BLOCK, _D), lambda i: (i, 0))],
        out_specs=pl.BlockSpec((1, _D), lambda i: (0, 0)),
        out_shape=jax.ShapeDtypeStruct((1, _D), jnp.float32),
        scratch_shapes=[pltpu.VMEM((8, _D), jnp.float32)],
    )(x)


# speedup vs baseline: 2.2057x; 1.2552x over previous
import functools

import jax
import jax.numpy as jnp
from jax.experimental import pallas as pl
from jax.experimental.pallas import tpu as pltpu

_N = 100000
_D = 128
_BLOCK = 10000


def _sum_body(x_ref, o_ref, acc_ref):
    i = pl.program_id(0)

    @pl.when(i == 0)
    def _init():
        acc_ref[...] = jnp.zeros_like(acc_ref)

    ones = jnp.ones((1, _BLOCK), jnp.float32)
    acc_ref[...] += jnp.dot(ones, x_ref[...],
                            preferred_element_type=jnp.float32)

    @pl.when(i == pl.num_programs(0) - 1)
    def _finish():
        o_ref[...] = acc_ref[...]


def kernel(x):
    grid = _N // _BLOCK
    return pl.pallas_call(
        _sum_body,
        grid=(grid,),
        in_specs=[pl.BlockSpec((_BLOCK, _D), lambda i: (i, 0))],
        out_specs=pl.BlockSpec((1, _D), lambda i: (0, 0)),
        out_shape=jax.ShapeDtypeStruct((1, _D), jnp.float32),
        scratch_shapes=[pltpu.VMEM((1, _D), jnp.float32)],
    )(x)
